# baseline (speedup 1.0000x reference)
import functools

import jax
import jax.numpy as jnp
from jax import lax
from jax.experimental import pallas as pl
from jax.experimental.pallas import tpu as pltpu
from jax.experimental.pallas import tpu_sc as plsc


def _make_selector(B, S, D):
    R = S // 2
    G = B * R
    NW = 32
    rows_per_w = G // NW

    mesh = plsc.VectorSubcoreMesh(core_axis_name="c", subcore_axis_name="s")

    @functools.partial(
        pl.kernel,
        mesh=mesh,
        out_type=jax.ShapeDtypeStruct((G, D), jnp.float32),
    )
    def run(x_hbm, out_hbm):
        wid = lax.axis_index("s") * 2 + lax.axis_index("c")
        base = wid * rows_per_w
        pltpu.sync_copy(
            x_hbm.at[pl.ds(base, rows_per_w), pl.ds(D, D)],
            out_hbm.at[pl.ds(base, rows_per_w), :],
        )

    return run


def kernel(inputs):
    B, S, D = inputs.shape
    x2 = inputs.reshape(B * (S // 2), 2 * D)
    out = _make_selector(B, S, D)(x2)
    return out.reshape(B, S // 2, D)


# speedup vs baseline: 10.7980x; 10.7980x over previous
import functools

import jax
import jax.numpy as jnp
from jax import lax
from jax.experimental import pallas as pl
from jax.experimental.pallas import tpu as pltpu
from jax.experimental.pallas import tpu_sc as plsc


def _make_selector(B, S, D):
    R = S // 2
    G = B * R
    NW = 32
    rows_per_w = G // NW

    C = 16
    NBUF = 4
    n_chunks = rows_per_w // C

    mesh = plsc.VectorSubcoreMesh(core_axis_name="c", subcore_axis_name="s")

    @functools.partial(
        pl.kernel,
        mesh=mesh,
        out_type=jax.ShapeDtypeStruct((G, D), jnp.float32),
        scratch_types=(
            [pltpu.VMEM((C, D), jnp.float32) for _ in range(NBUF)]
            + [pltpu.SemaphoreType.DMA for _ in range(2 * NBUF)]
        ),
    )
    def run(x_hbm, out_hbm, *scratch):
        bufs = scratch[:NBUF]
        lsems = scratch[NBUF:2 * NBUF]
        ssems = scratch[2 * NBUF:]
        wid = lax.axis_index("s") * 2 + lax.axis_index("c")
        base = wid * rows_per_w

        def load(g):
            b = g % NBUF
            return pltpu.async_copy(
                x_hbm.at[pl.ds(base + g * C, C), pl.ds(D, D)],
                bufs[b], lsems[b])

        def store(g):
            b = g % NBUF
            return pltpu.async_copy(
                bufs[b], out_hbm.at[pl.ds(base + g * C, C), :], ssems[b])

        ld = {0: load(0)}
        st = {}
        for g in range(n_chunks):
            nxt = g + 1
            if nxt < n_chunks:
                if nxt >= NBUF:
                    st[nxt - NBUF].wait()
                ld[nxt] = load(nxt)
            ld[g].wait()
            st[g] = store(g)
        for g in range(max(0, n_chunks - NBUF), n_chunks):
            st[g].wait()

    return run


def kernel(inputs):
    B, S, D = inputs.shape
    x2 = inputs.reshape(B * (S // 2), 2 * D)
    out = _make_selector(B, S, D)(x2)
    return out.reshape(B, S // 2, D)
